# baseline (speedup 1.0000x reference)
import functools

import jax
import jax.numpy as jnp
from jax import lax
from jax.experimental import pallas as pl
from jax.experimental.pallas import tpu as pltpu
from jax.experimental.pallas import tpu_sc as plsc

BATCH = 16384
EMB = 64
ROW = 2 * EMB
L = 16
NC = 2
NS = 16
NW = NC * NS
BPW = BATCH // NW
CH = 128
HB = 256
NH = BPW // HB

_mesh = plsc.VectorSubcoreMesh(core_axis_name="c", subcore_axis_name="s")


@functools.partial(
    pl.kernel,
    mesh=_mesh,
    out_type=jax.ShapeDtypeStruct((BATCH,), jnp.float32),
    compiler_params=pltpu.CompilerParams(use_tc_tiling_on_sc=False),
    scratch_types=[
        pltpu.VMEM((BPW,), jnp.int32),
        pltpu.VMEM((BPW,), jnp.int32),
        pltpu.VMEM((BPW,), jnp.int32),
        pltpu.VMEM((BPW,), jnp.int32),
        pltpu.VMEM((BPW,), jnp.float32),
        pltpu.VMEM((BPW,), jnp.float32),
        pltpu.VMEM((HB, ROW), jnp.float32),
        pltpu.VMEM((HB, ROW), jnp.float32),
        pltpu.VMEM((BPW,), jnp.float32),
        pltpu.VMEM((BPW,), jnp.float32),
        pltpu.VMEM((BPW,), jnp.float32),
        pltpu.SemaphoreType.DMA,
        pltpu.SemaphoreType.DMA,
    ],
)
def _mf_sc(u1_hbm, i1_hbm, ub_hbm, ib_hbm, ue2_hbm, ie2_hbm, out_hbm,
           uraw, iraw, urow, irow, upar, ipar, uev, iev, ubv, ibv, outv,
           gsem, bsem):
    wid = lax.axis_index("s") * NC + lax.axis_index("c")
    base = wid * BPW

    pltpu.sync_copy(u1_hbm.at[pl.ds(base, BPW)], uraw)
    pltpu.sync_copy(i1_hbm.at[pl.ds(base, BPW)], iraw)

    def split(g, carry):
        d = pl.ds(g * L, L)
        uv = uraw[d]
        iv = iraw[d]
        urow[d] = lax.shift_right_logical(uv, 1)
        irow[d] = lax.shift_right_logical(iv, 1)
        upar[d] = lax.bitwise_and(uv, 1).astype(jnp.float32)
        ipar[d] = lax.bitwise_and(iv, 1).astype(jnp.float32)
        return carry

    lax.fori_loop(0, BPW // L, split, 0)

    bias_copies = []
    for j in range(BPW // CH):
        d = pl.ds(j * CH, CH)
        bias_copies.append(
            pltpu.async_copy(ub_hbm.at[uraw.at[d]], ubv.at[d], bsem))
        bias_copies.append(
            pltpu.async_copy(ib_hbm.at[iraw.at[d]], ibv.at[d], bsem))
    for c in bias_copies:
        c.wait()

    ii = lax.iota(jnp.int32, L)
    _dnums = lax.GatherDimensionNumbers(
        offset_dims=(), collapsed_slice_dims=(0,), start_index_map=(0,))

    def lane_perm(x, idx):
        return lax.gather(x, idx[:, None], _dnums, (1,),
                          mode=lax.GatherScatterMode.PROMISE_IN_BOUNDS)

    def lane_sum(x):
        for k in (1, 2, 4, 8):
            x = x + lane_perm(x, ii ^ k)
        return x

    for h in range(NH):
        h0 = h * HB
        copies = []
        for j in range(HB // CH):
            s = pl.ds(h0 + j * CH, CH)
            d = pl.ds(j * CH, CH)
            copies.append(
                pltpu.async_copy(ue2_hbm.at[urow.at[s]], uev.at[d], gsem))
            copies.append(
                pltpu.async_copy(ie2_hbm.at[irow.at[s]], iev.at[d], gsem))
        for c in copies:
            c.wait()

        def group(g, carry):
            r0 = g * L
            b = pl.ds(h0 + r0, L)
            acc = ubv[b] + ibv[b]
            upv = upar[b]
            ipv = ipar[b]
            for r in range(L):
                rr = jnp.full((L,), r, jnp.int32)
                uf = lane_perm(upv, rr)
                vf = lane_perm(ipv, rr)
                q = None
                for c in range(EMB // L):
                    u0 = uev[r0 + r, pl.ds(c * L, L)]
                    u1 = uev[r0 + r, pl.ds(EMB + c * L, L)]
                    v0 = iev[r0 + r, pl.ds(c * L, L)]
                    v1 = iev[r0 + r, pl.ds(EMB + c * L, L)]
                    uu = u0 + uf * (u1 - u0)
                    vv = v0 + vf * (v1 - v0)
                    q = uu * vv if q is None else q + uu * vv
                s = lane_sum(q)
                acc = jnp.where(ii == r, acc + s, acc)
            outv[pl.ds(h0 + r0, L)] = acc
            return carry

        lax.fori_loop(0, HB // L, group, 0)

    pltpu.sync_copy(outv, out_hbm.at[pl.ds(base, BPW)])


def kernel(user, item, user_biases, item_biases, user_emb, item_emb):
    u1 = user.astype(jnp.int32)
    i1 = item.astype(jnp.int32)
    ub = user_biases.reshape(-1)
    ib = item_biases.reshape(-1)
    ue2 = user_emb.reshape(-1, ROW)
    ie2 = item_emb.reshape(-1, ROW)
    return _mf_sc(u1, i1, ub, ib, ue2, ie2)


# speedup vs baseline: 1.0062x; 1.0062x over previous
import functools

import jax
import jax.numpy as jnp
from jax import lax
from jax.experimental import pallas as pl
from jax.experimental.pallas import tpu as pltpu
from jax.experimental.pallas import tpu_sc as plsc

BATCH = 16384
EMB = 64
L = 16
NC = 2
NS = 16
NW = NC * NS
BPW = BATCH // NW
CH = 128
NCH = BPW // CH

_mesh = plsc.VectorSubcoreMesh(core_axis_name="c", subcore_axis_name="s")


@functools.partial(
    pl.kernel,
    mesh=_mesh,
    out_type=jax.ShapeDtypeStruct((BATCH,), jnp.float32),
    compiler_params=pltpu.CompilerParams(use_tc_tiling_on_sc=False),
    scratch_types=[
        pltpu.VMEM((NCH, CH), jnp.int32),
        pltpu.VMEM((NCH, CH), jnp.int32),
        pltpu.VMEM((BPW, EMB), jnp.float32),
        pltpu.VMEM((BPW, EMB), jnp.float32),
        pltpu.VMEM((BPW,), jnp.float32),
        pltpu.VMEM((BPW,), jnp.float32),
        pltpu.VMEM((BPW,), jnp.float32),
        pltpu.SemaphoreType.DMA,
    ],
)
def _mf_sc(user_hbm, item_hbm, ub_hbm, ib_hbm, ue_hbm, ie_hbm, out_hbm,
           uidx, iidx, uev, iev, ubv, ibv, outv, sem):
    wid = lax.axis_index("s") * NC + lax.axis_index("c")
    base = wid * BPW
    cbase = wid * NCH

    pltpu.sync_copy(user_hbm.at[pl.ds(cbase, NCH)], uidx)
    pltpu.sync_copy(item_hbm.at[pl.ds(cbase, NCH)], iidx)

    copies = []
    for j in range(NCH):
        d = pl.ds(j * CH, CH)
        copies.append(pltpu.async_copy(ue_hbm.at[uidx.at[j]], uev.at[d], sem))
        copies.append(pltpu.async_copy(ie_hbm.at[iidx.at[j]], iev.at[d], sem))
        copies.append(pltpu.async_copy(ub_hbm.at[uidx.at[j]], ubv.at[d], sem))
        copies.append(pltpu.async_copy(ib_hbm.at[iidx.at[j]], ibv.at[d], sem))
    for c in copies:
        c.wait()

    ii = lax.iota(jnp.int32, L)
    _dnums = lax.GatherDimensionNumbers(
        offset_dims=(), collapsed_slice_dims=(0,), start_index_map=(0,))

    def lane_perm(x, idx):
        return lax.gather(x, idx[:, None], _dnums, (1,),
                          mode=lax.GatherScatterMode.PROMISE_IN_BOUNDS)

    def lane_sum(x):
        for k in (1, 2, 4, 8):
            x = x + lane_perm(x, ii ^ k)
        return x

    def group(g, carry):
        r0 = g * L
        acc = ubv[pl.ds(r0, L)] + ibv[pl.ds(r0, L)]
        for r in range(L):
            q = None
            for cidx in range(EMB // L):
                u = uev[r0 + r, pl.ds(cidx * L, L)]
                v = iev[r0 + r, pl.ds(cidx * L, L)]
                q = u * v if q is None else q + u * v
            s = lane_sum(q)
            acc = jnp.where(ii == r, acc + s, acc)
        outv[pl.ds(r0, L)] = acc
        return carry

    lax.fori_loop(0, BPW // L, group, 0)

    pltpu.sync_copy(outv, out_hbm.at[pl.ds(base, BPW)])


def kernel(user, item, user_biases, item_biases, user_emb, item_emb):
    user2 = user.astype(jnp.int32).reshape(NW * NCH, CH)
    item2 = item.astype(jnp.int32).reshape(NW * NCH, CH)
    ub = user_biases.reshape(-1)
    ib = item_biases.reshape(-1)
    return _mf_sc(user2, item2, ub, ib, user_emb, item_emb)
